# Initial kernel scaffold; baseline (speedup 1.0000x reference)
import jax
import jax.numpy as jnp
from jax.experimental import pallas as pl


def kernel(W, edge_index, W1a, b1a, W1b, b1b, W2a, b2a, W2b, b2b, Wr1, br1, Wr2, br2, pe_W, pe_b, head_W, head_b):
    raise NotImplementedError("write your pallas kernel here")


# speedup vs baseline: 7.3039x; 7.3039x over previous
import functools

import jax
import jax.numpy as jnp
from jax import lax
from jax.experimental import pallas as pl
from jax.experimental.pallas import tpu as pltpu
from jax.experimental.pallas import tpu_sc as plsc

NC = 2
NS = 16
LANES = 128
GPC = 4

BR = 400


def _seg_sum_sc(n_pad, e_groups, feat):
  rpt = n_pad // NS
  gpt = e_groups // NS
  nch = gpt // GPC

  mesh = plsc.VectorSubcoreMesh(core_axis_name="c", subcore_axis_name="s")

  @functools.partial(
      pl.kernel,
      out_type=jax.ShapeDtypeStruct((NC, n_pad, feat), jnp.float32),
      mesh=mesh,
      scratch_types=[
          pltpu.VMEM_SHARED((n_pad, feat), jnp.float32),
          pltpu.VMEM((GPC, LANES), jnp.int32),
          pltpu.VMEM((GPC, LANES), jnp.int32),
          pltpu.VMEM((GPC * LANES, feat), jnp.float32),
          pltpu.SemaphoreType.DMA,
      ],
      compiler_params=pltpu.CompilerParams(use_tc_tiling_on_sc=False),
  )
  def seg_sum(tab, srcg, dstg, zeros, out, acc, src_v, dst_v, rows_v, sem):
    c = lax.axis_index("c")
    s = lax.axis_index("s")
    r0 = s * rpt
    pltpu.sync_copy(zeros, acc.at[pl.ds(r0, rpt)])
    plsc.subcore_barrier()

    base = s * gpt

    def chunk(i, carry):
      g0 = base + i * GPC
      pltpu.sync_copy(srcg.at[pl.ds(g0, GPC)], src_v)
      pltpu.sync_copy(dstg.at[pl.ds(g0, GPC)], dst_v)
      copies = []
      for j in range(GPC):
        copies.append(pltpu.async_copy(
            tab.at[c].at[src_v.at[j]], rows_v.at[pl.ds(j * LANES, LANES)],
            sem))
      for cp in copies:
        cp.wait()
      for j in range(GPC):
        pltpu.sync_copy(rows_v.at[pl.ds(j * LANES, LANES)],
                        acc.at[dst_v.at[j]], add=True)
      return carry

    lax.fori_loop(0, nch, chunk, 0)
    plsc.subcore_barrier()
    pltpu.sync_copy(acc.at[pl.ds(r0, rpt)], out.at[c, pl.ds(r0, rpt)])

  return seg_sum


def _mlp1_body(w2, a2, W1a, b1a, W1b, b1b, h2):
  x = jnp.concatenate([w2[0] + a2[0], w2[1] + a2[1]], axis=1)
  t = jnp.maximum(jnp.dot(x, W1a[...]) + b1a[...], 0.0)
  h = jnp.maximum(jnp.dot(t, W1b[...]) + b1b[...], 0.0)
  h2[0] = h[:, :32]
  h2[1] = h[:, 32:]


def _mlp2_body(h2, a2, W2a, b2a, W2b, b2b, Wr1, br1, Wr2, br2,
               peW, peb, hW, hb, out):
  x = jnp.concatenate([h2[0] + a2[0], h2[1] + a2[1]], axis=1)
  h = jnp.dot(jnp.maximum(jnp.dot(x, W2a[...]) + b2a[...], 0.0),
              W2b[...]) + b2b[...]
  pe = jnp.dot(jnp.maximum(jnp.dot(h, Wr1[...]) + br1[...], 0.0),
               Wr2[...]) + br2[...]
  emb = jnp.dot(pe, peW[...]) + peb[...]
  out[...] = jnp.dot(emb, hW[...]) + hb[...]


def _full(i):
  return (0, 0)


def kernel(W, edge_index, W1a, b1a, W1b, b1b, W2a, b2a, W2b, b2b,
           Wr1, br1, Wr2, br2, pe_W, pe_b, head_W, head_b):
  n, m = W.shape
  e = edge_index.shape[1]
  hid = W1b.shape[1]
  pe_dims = W2b.shape[1]
  channels = pe_W.shape[1]
  out_dim = head_W.shape[1]

  rpt = -(-(n + 1) // NS)
  rpt += (-rpt) % 8
  n_pad = NS * rpt
  egrp = -(-e // LANES)
  egrp = NS * GPC * (-(-egrp // (NS * GPC)))
  e_pad = egrp * LANES

  src = edge_index[0]
  dst = edge_index[1]
  pad_e = e_pad - e
  srcg = jnp.concatenate([src, jnp.zeros((pad_e,), jnp.int32)])
  srcg = srcg.reshape(egrp, LANES)
  dstg = jnp.concatenate([dst, jnp.full((pad_e,), n, jnp.int32)])
  dstg = dstg.reshape(egrp, LANES)

  m_half = 16 * (-(-(-(-m // 2)) // 16))
  m_pad = 2 * m_half
  w_tab = jnp.pad(W, ((0, n_pad - n), (0, m_pad - m)))
  w_tab = w_tab.reshape(n_pad, 2, m_half).transpose(1, 0, 2)

  zeros_a = jnp.zeros((rpt, m_half), jnp.float32)
  zeros_c = jnp.zeros((rpt, hid // 2), jnp.float32)

  agg1 = _seg_sum_sc(n_pad, egrp, m_half)(w_tab, srcg, dstg, zeros_a)

  nb = -(-n // BR)
  W1a_p = jnp.pad(W1a, ((0, m_pad - m), (0, 0)))
  h1 = pl.pallas_call(
      _mlp1_body,
      grid=(nb,),
      in_specs=[
          pl.BlockSpec((2, BR, m_half), lambda i: (0, i, 0)),
          pl.BlockSpec((2, BR, m_half), lambda i: (0, i, 0)),
          pl.BlockSpec((m_pad, hid), _full),
          pl.BlockSpec((1, hid), _full),
          pl.BlockSpec((hid, hid), _full),
          pl.BlockSpec((1, hid), _full),
      ],
      out_specs=pl.BlockSpec((2, BR, hid // 2), lambda i: (0, i, 0)),
      out_shape=jax.ShapeDtypeStruct((2, n_pad, hid // 2), jnp.float32),
  )(w_tab, agg1, W1a_p, b1a.reshape(1, hid), W1b, b1b.reshape(1, hid))

  agg2 = _seg_sum_sc(n_pad, egrp, hid // 2)(h1, srcg, dstg, zeros_c)

  out = pl.pallas_call(
      _mlp2_body,
      grid=(nb,),
      in_specs=[
          pl.BlockSpec((2, BR, hid // 2), lambda i: (0, i, 0)),
          pl.BlockSpec((2, BR, hid // 2), lambda i: (0, i, 0)),
          pl.BlockSpec((hid, hid), _full),
          pl.BlockSpec((1, hid), _full),
          pl.BlockSpec((hid, pe_dims), _full),
          pl.BlockSpec((1, pe_dims), _full),
          pl.BlockSpec((pe_dims, hid), _full),
          pl.BlockSpec((1, hid), _full),
          pl.BlockSpec((hid, pe_dims), _full),
          pl.BlockSpec((1, pe_dims), _full),
          pl.BlockSpec((pe_dims, channels), _full),
          pl.BlockSpec((1, channels), _full),
          pl.BlockSpec((channels, out_dim), _full),
          pl.BlockSpec((1, out_dim), _full),
      ],
      out_specs=pl.BlockSpec((BR, out_dim), lambda i: (i, 0)),
      out_shape=jax.ShapeDtypeStruct((nb * BR, out_dim), jnp.float32),
  )(h1, agg2, W2a, b2a.reshape(1, hid), W2b, b2b.reshape(1, pe_dims),
    Wr1, br1.reshape(1, hid), Wr2, br2.reshape(1, pe_dims),
    pe_W, pe_b.reshape(1, channels), head_W, head_b.reshape(1, out_dim))

  return out[:n]


# speedup vs baseline: 8.1955x; 1.1221x over previous
import functools

import jax
import jax.numpy as jnp
from jax import lax
from jax.experimental import pallas as pl
from jax.experimental.pallas import tpu as pltpu
from jax.experimental.pallas import tpu_sc as plsc

NC = 2
NS = 16
LANES = 128

BR = 400


def _seg_sum_sc(n_pad, e_groups, feat, gpc):
  rpt = n_pad // NS
  gpt = e_groups // NS
  npairs = gpt // (2 * gpc)
  rows = gpc * LANES

  mesh = plsc.VectorSubcoreMesh(core_axis_name="c", subcore_axis_name="s")

  @functools.partial(
      pl.kernel,
      out_type=jax.ShapeDtypeStruct((NC, n_pad, feat), jnp.float32),
      mesh=mesh,
      scratch_types=[
          pltpu.VMEM_SHARED((n_pad, feat), jnp.float32),
          [pltpu.VMEM((gpc, LANES), jnp.int32)] * 2,
          [pltpu.VMEM((gpc, LANES), jnp.int32)] * 2,
          [pltpu.VMEM((rows, feat), jnp.float32)] * 2,
          [pltpu.SemaphoreType.DMA] * 2,
          [pltpu.SemaphoreType.DMA] * 2,
      ],
      compiler_params=pltpu.CompilerParams(use_tc_tiling_on_sc=False),
  )
  def seg_sum(tab, srcg, dstg, zeros, out, acc, src_v, dst_v, rows_v,
              gsem, ssem):
    c = lax.axis_index("c")
    s = lax.axis_index("s")
    r0 = s * rpt
    pltpu.sync_copy(zeros, acc.at[pl.ds(r0, rpt)])
    plsc.subcore_barrier()

    base = s * gpt

    def fire_gathers(p, g0):
      pltpu.sync_copy(srcg.at[pl.ds(g0, gpc)], src_v[p])
      pltpu.sync_copy(dstg.at[pl.ds(g0, gpc)], dst_v[p])
      for j in range(gpc):
        pltpu.async_copy(tab.at[c].at[src_v[p].at[j]],
                         rows_v[p].at[pl.ds(j * LANES, LANES)], gsem[p])

    for p in range(2):
      fire_gathers(p, base + p * gpc)

    def pair(k, carry):
      for p in range(2):
        for j in range(gpc):
          pltpu.make_async_copy(
              tab.at[c].at[src_v[p].at[j]],
              rows_v[p].at[pl.ds(j * LANES, LANES)], gsem[p]).wait()
        for j in range(gpc):
          pltpu.async_copy(rows_v[p].at[pl.ds(j * LANES, LANES)],
                           acc.at[dst_v[p].at[j]], ssem[p], add=True)

        @pl.when(k < npairs - 1)
        def _():
          for j in range(gpc):
            pltpu.make_async_copy(rows_v[p].at[pl.ds(j * LANES, LANES)],
                                  acc.at[dst_v[p].at[j]], ssem[p]).wait()
          fire_gathers(p, base + (2 * k + p + 2) * gpc)
      return carry

    lax.fori_loop(0, npairs, pair, 0)
    for p in range(2):
      for j in range(gpc):
        pltpu.make_async_copy(rows_v[p].at[pl.ds(j * LANES, LANES)],
                              acc.at[dst_v[p].at[j]], ssem[p]).wait()
    plsc.subcore_barrier()
    pltpu.sync_copy(acc.at[pl.ds(r0, rpt)], out.at[c, pl.ds(r0, rpt)])

  return seg_sum


def _mlp1_body(w2, a2, W1a, b1a, W1b, b1b, h2):
  x = jnp.concatenate([w2[0] + a2[0], w2[1] + a2[1]], axis=1)
  t = jnp.maximum(jnp.dot(x, W1a[...]) + b1a[...], 0.0)
  h = jnp.maximum(jnp.dot(t, W1b[...]) + b1b[...], 0.0)
  h2[0] = h[:, :32]
  h2[1] = h[:, 32:]


def _mlp2_body(h2, a2, W2a, b2a, W2b, b2b, Wr1, br1, Wr2, br2,
               peW, peb, hW, hb, out):
  x = jnp.concatenate([h2[0] + a2[0], h2[1] + a2[1]], axis=1)
  h = jnp.dot(jnp.maximum(jnp.dot(x, W2a[...]) + b2a[...], 0.0),
              W2b[...]) + b2b[...]
  pe = jnp.dot(jnp.maximum(jnp.dot(h, Wr1[...]) + br1[...], 0.0),
               Wr2[...]) + br2[...]
  emb = jnp.dot(pe, peW[...]) + peb[...]
  out[...] = jnp.dot(emb, hW[...]) + hb[...]


def _full(i):
  return (0, 0)


def kernel(W, edge_index, W1a, b1a, W1b, b1b, W2a, b2a, W2b, b2b,
           Wr1, br1, Wr2, br2, pe_W, pe_b, head_W, head_b):
  n, m = W.shape
  e = edge_index.shape[1]
  hid = W1b.shape[1]
  pe_dims = W2b.shape[1]
  channels = pe_W.shape[1]
  out_dim = head_W.shape[1]

  rpt = -(-(n + 1) // NS)
  rpt += (-rpt) % 8
  n_pad = NS * rpt
  gpt = -(-(-(-e // LANES)) // NS)
  gpt = 12 * (-(-gpt // 12))
  egrp = NS * gpt
  e_pad = egrp * LANES

  src = edge_index[0]
  dst = edge_index[1]
  pad_e = e_pad - e
  srcg = jnp.concatenate([src, jnp.zeros((pad_e,), jnp.int32)])
  srcg = srcg.reshape(egrp, LANES)
  dstg = jnp.concatenate([dst, jnp.full((pad_e,), n, jnp.int32)])
  dstg = dstg.reshape(egrp, LANES)

  m_half = 16 * (-(-(-(-m // 2)) // 16))
  m_pad = 2 * m_half
  w_tab = jnp.pad(W, ((0, n_pad - n), (0, m_pad - m)))
  w_tab = w_tab.reshape(n_pad, 2, m_half).transpose(1, 0, 2)

  zeros_a = jnp.zeros((rpt, m_half), jnp.float32)
  zeros_c = jnp.zeros((rpt, hid // 2), jnp.float32)

  agg1 = _seg_sum_sc(n_pad, egrp, m_half, 6)(w_tab, srcg, dstg, zeros_a)

  nb = -(-n // BR)
  W1a_p = jnp.pad(W1a, ((0, m_pad - m), (0, 0)))
  h1 = pl.pallas_call(
      _mlp1_body,
      grid=(nb,),
      in_specs=[
          pl.BlockSpec((2, BR, m_half), lambda i: (0, i, 0)),
          pl.BlockSpec((2, BR, m_half), lambda i: (0, i, 0)),
          pl.BlockSpec((m_pad, hid), _full),
          pl.BlockSpec((1, hid), _full),
          pl.BlockSpec((hid, hid), _full),
          pl.BlockSpec((1, hid), _full),
      ],
      out_specs=pl.BlockSpec((2, BR, hid // 2), lambda i: (0, i, 0)),
      out_shape=jax.ShapeDtypeStruct((2, n_pad, hid // 2), jnp.float32),
  )(w_tab, agg1, W1a_p, b1a.reshape(1, hid), W1b, b1b.reshape(1, hid))

  agg2 = _seg_sum_sc(n_pad, egrp, hid // 2, 3)(h1, srcg, dstg, zeros_c)

  out = pl.pallas_call(
      _mlp2_body,
      grid=(nb,),
      in_specs=[
          pl.BlockSpec((2, BR, hid // 2), lambda i: (0, i, 0)),
          pl.BlockSpec((2, BR, hid // 2), lambda i: (0, i, 0)),
          pl.BlockSpec((hid, hid), _full),
          pl.BlockSpec((1, hid), _full),
          pl.BlockSpec((hid, pe_dims), _full),
          pl.BlockSpec((1, pe_dims), _full),
          pl.BlockSpec((pe_dims, hid), _full),
          pl.BlockSpec((1, hid), _full),
          pl.BlockSpec((hid, pe_dims), _full),
          pl.BlockSpec((1, pe_dims), _full),
          pl.BlockSpec((pe_dims, channels), _full),
          pl.BlockSpec((1, channels), _full),
          pl.BlockSpec((channels, out_dim), _full),
          pl.BlockSpec((1, out_dim), _full),
      ],
      out_specs=pl.BlockSpec((BR, out_dim), lambda i: (i, 0)),
      out_shape=jax.ShapeDtypeStruct((nb * BR, out_dim), jnp.float32),
  )(h1, agg2, W2a, b2a.reshape(1, hid), W2b, b2b.reshape(1, pe_dims),
    Wr1, br1.reshape(1, hid), Wr2, br2.reshape(1, pe_dims),
    pe_W, pe_b.reshape(1, channels), head_W, head_b.reshape(1, out_dim))

  return out[:n]


# speedup vs baseline: 9.8199x; 1.1982x over previous
import functools

import jax
import jax.numpy as jnp
from jax import lax
from jax.experimental import pallas as pl
from jax.experimental.pallas import tpu as pltpu
from jax.experimental.pallas import tpu_sc as plsc

NC = 2
NS = 16
LANES = 128

BR = 400


def _seg_sum_sc(n_pad, e_groups, feat, gpc, tail_sizes):
  rpt = n_pad // NS
  gfloor = e_groups // NS
  grem = e_groups - NS * gfloor
  period = 2 * gpc
  assert sum(tail_sizes) >= period - 1 and max(tail_sizes) <= gpc

  mesh = plsc.VectorSubcoreMesh(core_axis_name="c", subcore_axis_name="s")

  @functools.partial(
      pl.kernel,
      out_type=jax.ShapeDtypeStruct((n_pad, 128), jnp.float32),
      mesh=mesh,
      scratch_types=[
          pltpu.VMEM_SHARED((n_pad, feat), jnp.float32),
          [pltpu.VMEM((gpc, LANES), jnp.int32)] * 2,
          [pltpu.VMEM((gpc, LANES), jnp.int32)] * 2,
          [pltpu.VMEM((gpc * LANES, feat), jnp.float32)] * 2,
          [pltpu.SemaphoreType.DMA] * 2,
          [pltpu.SemaphoreType.DMA] * 2,
      ],
      compiler_params=pltpu.CompilerParams(use_tc_tiling_on_sc=False),
  )
  def seg_sum(tab, srcg, dstg, zeros, out, acc, src_v, dst_v, rows_v,
              gsem, ssem):
    c = lax.axis_index("c")
    s = lax.axis_index("s")
    r0 = s * rpt
    pltpu.sync_copy(zeros.at[:, pl.ds(0, feat)], acc.at[pl.ds(r0, rpt)])
    plsc.subcore_barrier()

    base = s * gfloor + jnp.minimum(s, grem)
    gcnt = gfloor + jnp.where(s < grem, 1, 0)
    npairs = gcnt // period

    def fire_gathers(p, g0):
      pltpu.sync_copy(srcg.at[pl.ds(g0, gpc)], src_v[p])
      pltpu.sync_copy(dstg.at[pl.ds(g0, gpc)], dst_v[p])
      for j in range(gpc):
        pltpu.async_copy(tab.at[c].at[src_v[p].at[j]],
                         rows_v[p].at[pl.ds(j * LANES, LANES)], gsem[p])

    for p in range(2):
      fire_gathers(p, base + p * gpc)

    def pair(k, carry):
      for p in range(2):
        for j in range(gpc):
          pltpu.make_async_copy(
              tab.at[c].at[src_v[p].at[j]],
              rows_v[p].at[pl.ds(j * LANES, LANES)], gsem[p]).wait()
        for j in range(gpc):
          pltpu.async_copy(rows_v[p].at[pl.ds(j * LANES, LANES)],
                           acc.at[dst_v[p].at[j]], ssem[p], add=True)

        @pl.when(k < npairs - 1)
        def _():
          for j in range(gpc):
            pltpu.make_async_copy(rows_v[p].at[pl.ds(j * LANES, LANES)],
                                  acc.at[dst_v[p].at[j]], ssem[p]).wait()
          fire_gathers(p, base + (2 * k + p + 2) * gpc)
      return carry

    lax.fori_loop(0, npairs, pair, 0)
    for p in range(2):
      for j in range(gpc):
        pltpu.make_async_copy(rows_v[p].at[pl.ds(j * LANES, LANES)],
                              acc.at[dst_v[p].at[j]], ssem[p]).wait()

    rem = gcnt - npairs * period
    done = 0
    for b in tail_sizes:
      pred = rem - done >= b

      @pl.when(pred)
      def _():
        g = base + npairs * period + done
        pltpu.sync_copy(srcg.at[pl.ds(g, b)], src_v[0].at[pl.ds(0, b)])
        pltpu.sync_copy(dstg.at[pl.ds(g, b)], dst_v[0].at[pl.ds(0, b)])
        for j in range(b):
          pltpu.async_copy(tab.at[c].at[src_v[0].at[j]],
                           rows_v[0].at[pl.ds(j * LANES, LANES)], gsem[0])
        for j in range(b):
          pltpu.make_async_copy(
              tab.at[c].at[src_v[0].at[j]],
              rows_v[0].at[pl.ds(j * LANES, LANES)], gsem[0]).wait()
        for j in range(b):
          pltpu.sync_copy(rows_v[0].at[pl.ds(j * LANES, LANES)],
                          acc.at[dst_v[0].at[j]], add=True)

      done = jnp.where(pred, done + b, done)

    plsc.subcore_barrier()
    pltpu.sync_copy(acc.at[pl.ds(r0, rpt)],
                    out.at[pl.ds(r0, rpt), pl.ds(c * feat, feat)])

  return seg_sum


def _mlp1_body(w2, a, W1a, b1a, W1b, b1b, h1):
  cols = W1a.shape[0]
  x = jnp.concatenate([w2[0], w2[1]], axis=1) + a[:, :cols]
  t = jnp.maximum(jnp.dot(x, W1a[...]) + b1a[...], 0.0)
  h = jnp.maximum(jnp.dot(t, W1b[...]) + b1b[...], 0.0)
  h1[0] = h[:, :h.shape[1] // 2]
  h1[1] = h[:, h.shape[1] // 2:]


def _mlp2_body(h2, a, W2a, b2a, W2b, b2b, Wr1, br1, Wr2, br2,
               peW, peb, hW, hb, out):
  cols = W2a.shape[0]
  x = jnp.concatenate([h2[0], h2[1]], axis=1) + a[:, :cols]
  h = jnp.dot(jnp.maximum(jnp.dot(x, W2a[...]) + b2a[...], 0.0),
              W2b[...]) + b2b[...]
  pe = jnp.dot(jnp.maximum(jnp.dot(h, Wr1[...]) + br1[...], 0.0),
               Wr2[...]) + br2[...]
  emb = jnp.dot(pe, peW[...]) + peb[...]
  out[...] = jnp.dot(emb, hW[...]) + hb[...]


def _full(i):
  return (0, 0)


def kernel(W, edge_index, W1a, b1a, W1b, b1b, W2a, b2a, W2b, b2b,
           Wr1, br1, Wr2, br2, pe_W, pe_b, head_W, head_b):
  n, m = W.shape
  e = edge_index.shape[1]
  hid = W1b.shape[1]
  pe_dims = W2b.shape[1]
  channels = pe_W.shape[1]
  out_dim = head_W.shape[1]

  rpt = -(-(n + 1) // NS)
  rpt += (-rpt) % 8
  n_pad = NS * rpt
  m_half = 16 * (-(-(-(-m // 2)) // 16))

  egrp = e // LANES
  assert egrp * LANES == e, "edge count must be lane-aligned"
  srcg = edge_index[0].reshape(egrp, LANES)
  dstg = edge_index[1].reshape(egrp, LANES)

  w_pad = jnp.pad(W, ((0, n_pad - n), (0, 2 * m_half - m)))
  w_tab = w_pad.reshape(n_pad, 2, m_half).transpose(1, 0, 2)
  zeros = jnp.zeros((rpt, 128), jnp.float32)

  agg1 = _seg_sum_sc(n_pad, egrp, m_half, 4, (4, 2, 1))(
      w_tab, srcg, dstg, zeros)

  nb = -(-n // BR)
  W1a_p = jnp.pad(W1a, ((0, 2 * m_half - m), (0, 0)))
  h1 = pl.pallas_call(
      _mlp1_body,
      grid=(nb,),
      in_specs=[
          pl.BlockSpec((2, BR, m_half), lambda i: (0, i, 0)),
          pl.BlockSpec((BR, 128), lambda i: (i, 0)),
          pl.BlockSpec((2 * m_half, hid), _full),
          pl.BlockSpec((1, hid), _full),
          pl.BlockSpec((hid, hid), _full),
          pl.BlockSpec((1, hid), _full),
      ],
      out_specs=pl.BlockSpec((2, BR, hid // 2), lambda i: (0, i, 0)),
      out_shape=jax.ShapeDtypeStruct((2, n_pad, hid // 2), jnp.float32),
  )(w_tab, agg1, W1a_p, b1a.reshape(1, hid), W1b, b1b.reshape(1, hid))

  agg2 = _seg_sum_sc(n_pad, egrp, hid // 2, 3, (2, 2, 1))(
      h1, srcg, dstg, zeros)

  out = pl.pallas_call(
      _mlp2_body,
      grid=(nb,),
      in_specs=[
          pl.BlockSpec((2, BR, hid // 2), lambda i: (0, i, 0)),
          pl.BlockSpec((BR, 128), lambda i: (i, 0)),
          pl.BlockSpec((hid, hid), _full),
          pl.BlockSpec((1, hid), _full),
          pl.BlockSpec((hid, pe_dims), _full),
          pl.BlockSpec((1, pe_dims), _full),
          pl.BlockSpec((pe_dims, hid), _full),
          pl.BlockSpec((1, hid), _full),
          pl.BlockSpec((hid, pe_dims), _full),
          pl.BlockSpec((1, pe_dims), _full),
          pl.BlockSpec((pe_dims, channels), _full),
          pl.BlockSpec((1, channels), _full),
          pl.BlockSpec((channels, out_dim), _full),
          pl.BlockSpec((1, out_dim), _full),
      ],
      out_specs=pl.BlockSpec((BR, out_dim), lambda i: (i, 0)),
      out_shape=jax.ShapeDtypeStruct((nb * BR, out_dim), jnp.float32),
  )(h1, agg2, W2a, b2a.reshape(1, hid), W2b, b2b.reshape(1, pe_dims),
    Wr1, br1.reshape(1, hid), Wr2, br2.reshape(1, pe_dims),
    pe_W, pe_b.reshape(1, channels), head_W, head_b.reshape(1, out_dim))

  return out[:n]


# speedup vs baseline: 13.0461x; 1.3285x over previous
import functools

import jax
import jax.numpy as jnp
from jax import lax
from jax.experimental import pallas as pl
from jax.experimental.pallas import tpu as pltpu
from jax.experimental.pallas import tpu_sc as plsc

NC = 2
NS = 16
LANES = 128

BR = 2000


def _seg_sum_sc(n_pad, e_groups, feat, gpc, tail_sizes):
  rpt = n_pad // NS
  gfloor = e_groups // NS
  grem = e_groups - NS * gfloor
  period = 2 * gpc
  assert sum(tail_sizes) >= period - 1 and max(tail_sizes) <= gpc

  mesh = plsc.VectorSubcoreMesh(core_axis_name="c", subcore_axis_name="s")

  @functools.partial(
      pl.kernel,
      out_type=jax.ShapeDtypeStruct((n_pad, 128), jnp.float32),
      mesh=mesh,
      scratch_types=[
          pltpu.VMEM_SHARED((n_pad, feat), jnp.float32),
          [pltpu.VMEM((gpc, LANES), jnp.int32)] * 2,
          [pltpu.VMEM((gpc, LANES), jnp.int32)] * 2,
          [pltpu.VMEM((gpc * LANES, feat), jnp.float32)] * 2,
          [pltpu.SemaphoreType.DMA] * 2,
          [pltpu.SemaphoreType.DMA] * 2,
      ],
      compiler_params=pltpu.CompilerParams(use_tc_tiling_on_sc=False),
  )
  def seg_sum(tab, srcg, dstg, out, acc, src_v, dst_v, rows_v,
              gsem, ssem):
    c = lax.axis_index("c")
    s = lax.axis_index("s")
    r0 = s * rpt
    pltpu.sync_copy(tab.at[c, pl.ds(r0, rpt)], acc.at[pl.ds(r0, rpt)])
    plsc.subcore_barrier()

    base = s * gfloor + jnp.minimum(s, grem)
    gcnt = gfloor + jnp.where(s < grem, 1, 0)
    npairs = gcnt // period

    def fire_gathers(p, g0):
      pltpu.sync_copy(srcg.at[pl.ds(g0, gpc)], src_v[p])
      pltpu.sync_copy(dstg.at[pl.ds(g0, gpc)], dst_v[p])
      for j in range(gpc):
        pltpu.async_copy(tab.at[c].at[src_v[p].at[j]],
                         rows_v[p].at[pl.ds(j * LANES, LANES)], gsem[p])

    for p in range(2):
      fire_gathers(p, base + p * gpc)

    def pair(k, carry):
      for p in range(2):
        for j in range(gpc):
          pltpu.make_async_copy(
              tab.at[c].at[src_v[p].at[j]],
              rows_v[p].at[pl.ds(j * LANES, LANES)], gsem[p]).wait()
        for j in range(gpc):
          pltpu.async_copy(rows_v[p].at[pl.ds(j * LANES, LANES)],
                           acc.at[dst_v[p].at[j]], ssem[p], add=True)

        @pl.when(k < npairs - 1)
        def _():
          for j in range(gpc):
            pltpu.make_async_copy(rows_v[p].at[pl.ds(j * LANES, LANES)],
                                  acc.at[dst_v[p].at[j]], ssem[p]).wait()
          fire_gathers(p, base + (2 * k + p + 2) * gpc)
      return carry

    lax.fori_loop(0, npairs, pair, 0)
    for p in range(2):
      for j in range(gpc):
        pltpu.make_async_copy(rows_v[p].at[pl.ds(j * LANES, LANES)],
                              acc.at[dst_v[p].at[j]], ssem[p]).wait()

    rem = gcnt - npairs * period
    done = 0
    for b in tail_sizes:
      pred = rem - done >= b

      @pl.when(pred)
      def _():
        g = base + npairs * period + done
        pltpu.sync_copy(srcg.at[pl.ds(g, b)], src_v[0].at[pl.ds(0, b)])
        pltpu.sync_copy(dstg.at[pl.ds(g, b)], dst_v[0].at[pl.ds(0, b)])
        for j in range(b):
          pltpu.async_copy(tab.at[c].at[src_v[0].at[j]],
                           rows_v[0].at[pl.ds(j * LANES, LANES)], gsem[0])
        for j in range(b):
          pltpu.make_async_copy(
              tab.at[c].at[src_v[0].at[j]],
              rows_v[0].at[pl.ds(j * LANES, LANES)], gsem[0]).wait()
        for j in range(b):
          pltpu.sync_copy(rows_v[0].at[pl.ds(j * LANES, LANES)],
                          acc.at[dst_v[0].at[j]], add=True)

      done = jnp.where(pred, done + b, done)

    plsc.subcore_barrier()
    pltpu.sync_copy(acc.at[pl.ds(r0, rpt)],
                    out.at[pl.ds(r0, rpt), pl.ds(c * feat, feat)])

  return seg_sum


def _mlp1_body(a, W1a, b1a, W1b, b1b, h1):
  x = a[:, :W1a.shape[0]]
  t = jnp.maximum(jnp.dot(x, W1a[...]) + b1a[...], 0.0)
  h = jnp.maximum(jnp.dot(t, W1b[...]) + b1b[...], 0.0)
  h1[0] = h[:, :h.shape[1] // 2]
  h1[1] = h[:, h.shape[1] // 2:]


def _mlp2_body(a, W2a, b2a, W2b, b2b, Wr1, br1, Wr2, br2,
               peW, peb, hW, hb, out):
  x = a[:, :W2a.shape[0]]
  h = jnp.dot(jnp.maximum(jnp.dot(x, W2a[...]) + b2a[...], 0.0),
              W2b[...]) + b2b[...]
  pe = jnp.dot(jnp.maximum(jnp.dot(h, Wr1[...]) + br1[...], 0.0),
               Wr2[...]) + br2[...]
  emb = jnp.dot(pe, peW[...]) + peb[...]
  out[...] = jnp.dot(emb, hW[...]) + hb[...]


def _full(i):
  return (0, 0)


def kernel(W, edge_index, W1a, b1a, W1b, b1b, W2a, b2a, W2b, b2b,
           Wr1, br1, Wr2, br2, pe_W, pe_b, head_W, head_b):
  n, m = W.shape
  e = edge_index.shape[1]
  hid = W1b.shape[1]
  pe_dims = W2b.shape[1]
  channels = pe_W.shape[1]
  out_dim = head_W.shape[1]

  rpt = -(-(n + 1) // NS)
  rpt += (-rpt) % 8
  n_pad = NS * rpt
  m_half = 16 * (-(-(-(-m // 2)) // 16))

  egrp = e // LANES
  assert egrp * LANES == e, "edge count must be lane-aligned"
  srcg = edge_index[0].reshape(egrp, LANES)
  dstg = edge_index[1].reshape(egrp, LANES)

  w_pad = jnp.pad(W, ((0, n_pad - n), (0, 2 * m_half - m)))
  w_tab = w_pad.reshape(n_pad, 2, m_half).transpose(1, 0, 2)

  agg1 = _seg_sum_sc(n_pad, egrp, m_half, 4, (4, 2, 1))(w_tab, srcg, dstg)

  nb = -(-n // BR)
  W1a_p = jnp.pad(W1a, ((0, 2 * m_half - m), (0, 0)))
  h1 = pl.pallas_call(
      _mlp1_body,
      grid=(nb,),
      in_specs=[
          pl.BlockSpec((BR, 128), lambda i: (i, 0)),
          pl.BlockSpec((2 * m_half, hid), _full),
          pl.BlockSpec((1, hid), _full),
          pl.BlockSpec((hid, hid), _full),
          pl.BlockSpec((1, hid), _full),
      ],
      out_specs=pl.BlockSpec((2, BR, hid // 2), lambda i: (0, i, 0)),
      out_shape=jax.ShapeDtypeStruct((2, n_pad, hid // 2), jnp.float32),
  )(agg1, W1a_p, b1a.reshape(1, hid), W1b, b1b.reshape(1, hid))

  agg2 = _seg_sum_sc(n_pad, egrp, hid // 2, 3, (2, 2, 1))(h1, srcg, dstg)

  out = pl.pallas_call(
      _mlp2_body,
      grid=(nb,),
      in_specs=[
          pl.BlockSpec((BR, 128), lambda i: (i, 0)),
          pl.BlockSpec((hid, hid), _full),
          pl.BlockSpec((1, hid), _full),
          pl.BlockSpec((hid, pe_dims), _full),
          pl.BlockSpec((1, pe_dims), _full),
          pl.BlockSpec((pe_dims, hid), _full),
          pl.BlockSpec((1, hid), _full),
          pl.BlockSpec((hid, pe_dims), _full),
          pl.BlockSpec((1, pe_dims), _full),
          pl.BlockSpec((pe_dims, channels), _full),
          pl.BlockSpec((1, channels), _full),
          pl.BlockSpec((channels, out_dim), _full),
          pl.BlockSpec((1, out_dim), _full),
      ],
      out_specs=pl.BlockSpec((BR, out_dim), lambda i: (i, 0)),
      out_shape=jax.ShapeDtypeStruct((nb * BR, out_dim), jnp.float32),
  )(agg2, W2a, b2a.reshape(1, hid), W2b, b2b.reshape(1, pe_dims),
    Wr1, br1.reshape(1, hid), Wr2, br2.reshape(1, pe_dims),
    pe_W, pe_b.reshape(1, channels), head_W, head_b.reshape(1, out_dim))

  return out[:n]


# speedup vs baseline: 16.9417x; 1.2986x over previous
import functools

import jax
import jax.numpy as jnp
from jax import lax
from jax.experimental import pallas as pl
from jax.experimental.pallas import tpu as pltpu
from jax.experimental.pallas import tpu_sc as plsc

NC = 2
NS = 16
LANES = 128

BR = 2000


def _seg_sum_sc(n_pad, e_groups, feat, gpc, tail_sizes):
  rpt = n_pad // NS
  gfloor = e_groups // NS
  grem = e_groups - NS * gfloor
  period = 2 * gpc
  assert sum(tail_sizes) >= period - 1 and max(tail_sizes) <= gpc

  mesh = plsc.VectorSubcoreMesh(core_axis_name="c", subcore_axis_name="s")

  @functools.partial(
      pl.kernel,
      out_type=jax.ShapeDtypeStruct((n_pad, 128), jnp.float32),
      mesh=mesh,
      scratch_types=[
          pltpu.VMEM_SHARED((n_pad, feat), jnp.float32),
          [pltpu.VMEM((2, gpc, LANES), jnp.int32)] * 2,
          [pltpu.VMEM((gpc * LANES, feat), jnp.float32)] * 2,
          [pltpu.SemaphoreType.DMA] * 2,
          [pltpu.SemaphoreType.DMA] * 2,
      ],
      compiler_params=pltpu.CompilerParams(use_tc_tiling_on_sc=False),
  )
  def seg_sum(tab, eg, out, acc, sd_v, rows_v, gsem, ssem):
    c = lax.axis_index("c")
    s = lax.axis_index("s")
    r0 = s * rpt
    pltpu.sync_copy(tab.at[c, pl.ds(r0, rpt)], acc.at[pl.ds(r0, rpt)])
    plsc.subcore_barrier()

    base = s * gfloor + jnp.minimum(s, grem)
    gcnt = gfloor + jnp.where(s < grem, 1, 0)
    npairs = gcnt // period

    def drain(n_groups, sem, p, to_acc):
      nr = n_groups * LANES
      if to_acc:
        pltpu.make_async_copy(rows_v[p].at[pl.ds(0, nr)],
                              acc.at[pl.ds(0, nr)], sem).wait()
      else:
        pltpu.make_async_copy(tab.at[c].at[pl.ds(0, nr)],
                              rows_v[p].at[pl.ds(0, nr)], sem).wait()

    def fire_gathers(p, g0):
      pltpu.sync_copy(eg.at[:, pl.ds(g0, gpc)], sd_v[p])
      for j in range(gpc):
        pltpu.async_copy(tab.at[c].at[sd_v[p].at[0, j]],
                         rows_v[p].at[pl.ds(j * LANES, LANES)], gsem[p])

    for p in range(2):
      fire_gathers(p, base + p * gpc)

    def pair(k, carry):
      for p in range(2):
        drain(gpc, gsem[p], p, False)
        for j in range(gpc):
          pltpu.async_copy(rows_v[p].at[pl.ds(j * LANES, LANES)],
                           acc.at[sd_v[p].at[1, j]], ssem[p], add=True)

        @pl.when(k < npairs - 1)
        def _():
          drain(gpc, ssem[p], p, True)
          fire_gathers(p, base + (2 * k + p + 2) * gpc)
      return carry

    lax.fori_loop(0, npairs, pair, 0)
    for p in range(2):
      drain(gpc, ssem[p], p, True)

    rem = gcnt - npairs * period
    done = 0
    for b in tail_sizes:
      pred = rem - done >= b

      @pl.when(pred)
      def _():
        g = base + npairs * period + done
        pltpu.sync_copy(eg.at[:, pl.ds(g, b)], sd_v[0].at[:, pl.ds(0, b)])
        for j in range(b):
          pltpu.async_copy(tab.at[c].at[sd_v[0].at[0, j]],
                           rows_v[0].at[pl.ds(j * LANES, LANES)], gsem[0])
        drain(b, gsem[0], 0, False)
        for j in range(b):
          pltpu.sync_copy(rows_v[0].at[pl.ds(j * LANES, LANES)],
                          acc.at[sd_v[0].at[1, j]], add=True)

      done = jnp.where(pred, done + b, done)

    plsc.subcore_barrier()
    pltpu.sync_copy(acc.at[pl.ds(r0, rpt)],
                    out.at[pl.ds(r0, rpt), pl.ds(c * feat, feat)])

  return seg_sum


def _mlp1_body(a, W1a, b1a, W1b, b1b, h1):
  x = a[:, :W1a.shape[0]]
  t = jnp.maximum(jnp.dot(x, W1a[...]) + b1a[...], 0.0)
  h = jnp.maximum(jnp.dot(t, W1b[...]) + b1b[...], 0.0)
  h1[0] = h[:, :h.shape[1] // 2]
  h1[1] = h[:, h.shape[1] // 2:]


def _mlp2_body(a, W2a, b2a, W2b, b2b, Wr1, br1, Wr2, br2,
               peW, peb, hW, hb, out):
  x = a[:, :W2a.shape[0]]
  h = jnp.dot(jnp.maximum(jnp.dot(x, W2a[...]) + b2a[...], 0.0),
              W2b[...]) + b2b[...]
  pe = jnp.dot(jnp.maximum(jnp.dot(h, Wr1[...]) + br1[...], 0.0),
               Wr2[...]) + br2[...]
  emb = jnp.dot(pe, peW[...]) + peb[...]
  out[...] = jnp.dot(emb, hW[...]) + hb[...]


def _full(i):
  return (0, 0)


def kernel(W, edge_index, W1a, b1a, W1b, b1b, W2a, b2a, W2b, b2b,
           Wr1, br1, Wr2, br2, pe_W, pe_b, head_W, head_b):
  n, m = W.shape
  e = edge_index.shape[1]
  hid = W1b.shape[1]
  pe_dims = W2b.shape[1]
  channels = pe_W.shape[1]
  out_dim = head_W.shape[1]

  rpt = -(-(n + 1) // NS)
  rpt += (-rpt) % 8
  n_pad = NS * rpt
  m_half = 16 * (-(-(-(-m // 2)) // 16))

  egrp = e // LANES
  assert egrp * LANES == e, "edge count must be lane-aligned"
  eg = edge_index.reshape(2, egrp, LANES)

  w_pad = jnp.pad(W, ((0, n_pad - n), (0, 2 * m_half - m)))
  w_tab = w_pad.reshape(n_pad, 2, m_half).transpose(1, 0, 2)

  agg1 = _seg_sum_sc(n_pad, egrp, m_half, 8, (8, 4, 2, 1))(w_tab, eg)

  nb = -(-n // BR)
  W1a_p = jnp.pad(W1a, ((0, 2 * m_half - m), (0, 0)))
  h1 = pl.pallas_call(
      _mlp1_body,
      grid=(nb,),
      in_specs=[
          pl.BlockSpec((BR, 128), lambda i: (i, 0)),
          pl.BlockSpec((2 * m_half, hid), _full),
          pl.BlockSpec((1, hid), _full),
          pl.BlockSpec((hid, hid), _full),
          pl.BlockSpec((1, hid), _full),
      ],
      out_specs=pl.BlockSpec((2, BR, hid // 2), lambda i: (0, i, 0)),
      out_shape=jax.ShapeDtypeStruct((2, n_pad, hid // 2), jnp.float32),
  )(agg1, W1a_p, b1a.reshape(1, hid), W1b, b1b.reshape(1, hid))

  agg2 = _seg_sum_sc(n_pad, egrp, hid // 2, 3, (2, 2, 1))(h1, eg)

  out = pl.pallas_call(
      _mlp2_body,
      grid=(nb,),
      in_specs=[
          pl.BlockSpec((BR, 128), lambda i: (i, 0)),
          pl.BlockSpec((hid, hid), _full),
          pl.BlockSpec((1, hid), _full),
          pl.BlockSpec((hid, pe_dims), _full),
          pl.BlockSpec((1, pe_dims), _full),
          pl.BlockSpec((pe_dims, hid), _full),
          pl.BlockSpec((1, hid), _full),
          pl.BlockSpec((hid, pe_dims), _full),
          pl.BlockSpec((1, pe_dims), _full),
          pl.BlockSpec((pe_dims, channels), _full),
          pl.BlockSpec((1, channels), _full),
          pl.BlockSpec((channels, out_dim), _full),
          pl.BlockSpec((1, out_dim), _full),
      ],
      out_specs=pl.BlockSpec((BR, out_dim), lambda i: (i, 0)),
      out_shape=jax.ShapeDtypeStruct((nb * BR, out_dim), jnp.float32),
  )(agg2, W2a, b2a.reshape(1, hid), W2b, b2b.reshape(1, pe_dims),
    Wr1, br1.reshape(1, hid), Wr2, br2.reshape(1, pe_dims),
    pe_W, pe_b.reshape(1, channels), head_W, head_b.reshape(1, out_dim))

  return out[:n]
